# baseline (speedup 1.0000x reference)
import dataclasses
import functools

import jax
import jax.numpy as jnp
from jax import lax
from jax.experimental import pallas as pl
from jax.experimental.pallas import tpu as pltpu
from jax.experimental.pallas import tpu_sc as plsc

_NC = 19
_R = 128
_H = 512
_W = 512
_B = 4
_TCB = 3
_GPIX = _H * _W

_NW = 32
_HPAD = 368
_LANES = 16

_TC_PIX = _TCB * _GPIX
_CHUNK_B = _TC_PIX // _NW
_CHUNK_A = _GPIX // _NW
_RND = 4096


def _sc_compiler_params():
    cp = pltpu.CompilerParams()
    if "needs_layout_passes" in pltpu.CompilerParams.__dataclass_fields__:
        cp = dataclasses.replace(cp, needs_layout_passes=False)
    return cp


_MESH = plsc.VectorSubcoreMesh(core_axis_name="c", subcore_axis_name="s")


def _argmax_body(pred_ref, label_ref, comb_ref):
    x = pred_ref[0]
    best = x[0]
    bidx = jnp.zeros((_R, _W), jnp.int32)
    for c in range(1, _NC):
        xc = x[c]
        take = xc > best
        best = jnp.where(take, xc, best)
        bidx = jnp.where(take, c, bidx)
    comb_ref[0] = label_ref[0] * _NC + bidx


def _combined_index(pred, label):
    return pl.pallas_call(
        _argmax_body,
        grid=(_TCB, _H // _R),
        in_specs=[
            pl.BlockSpec((1, _NC, _R, _W), lambda b, r: (b, 0, r, 0)),
            pl.BlockSpec((1, _R, _W), lambda b, r: (b, r, 0)),
        ],
        out_specs=pl.BlockSpec((1, _R, _W), lambda b, r: (b, r, 0)),
        out_shape=jax.ShapeDtypeStruct((_TCB, _H, _W), jnp.int32),
    )(pred, label)


def _hist_accumulate(h16_v, lane_base, ones, comb_v, n):
    @plsc.parallel_loop(0, n, step=_LANES, unroll=8)
    def _accum(i):
        v = comb_v[pl.ds(i, _LANES)]
        plsc.addupdate_scatter(h16_v, [lane_base + v], ones)


def _hist_zero_and_reduce_defs(h16_v, hsum_v, zeros):
    @plsc.parallel_loop(0, _HPAD, step=_LANES, unroll=2)
    def _reduce(c):
        acc = zeros
        for l in range(_LANES):
            acc = acc + h16_v[pl.ds(l * _HPAD + c, _LANES)]
        hsum_v[pl.ds(c, _LANES)] = acc


def _sc_argmax_hist(predf, labelf):

    @functools.partial(
        pl.kernel,
        compiler_params=_sc_compiler_params(),
        out_type=jax.ShapeDtypeStruct((_NW, _HPAD), jnp.int32),
        mesh=_MESH,
        scratch_types=[
            pltpu.VMEM((_NC * _RND,), jnp.float32),
            pltpu.VMEM((_CHUNK_A,), jnp.int32),
            pltpu.VMEM((_LANES * _HPAD,), jnp.int32),
            pltpu.VMEM((_HPAD,), jnp.int32),
            pltpu.SemaphoreType.DMA,
            pltpu.SemaphoreType.DMA,
        ],
    )
    def amh_kernel(pred_hbm, lab_hbm, out_hbm, cls_v, lab_v, h16_v, hsum_v,
                   sem, lsem):
        wid = lax.axis_index("s") * 2 + lax.axis_index("c")
        base = wid * _CHUNK_A

        lcp = pltpu.async_copy(
            lab_hbm.at[pl.ds(_TCB * _GPIX + base, _CHUNK_A)], lab_v, lsem)

        lane = lax.iota(jnp.int32, _LANES)
        lane_base = lane * _HPAD
        ones = jnp.ones((_LANES,), jnp.int32)
        zeros = jnp.zeros((_LANES,), jnp.int32)

        @plsc.parallel_loop(0, _LANES * _HPAD, step=_LANES, unroll=8)
        def _zero(i):
            h16_v[pl.ds(i, _LANES)] = zeros

        lcp.wait()

        for rnd in range(_CHUNK_A // _RND):
            pixbase = base + rnd * _RND
            copies = []
            for c in range(_NC):
                copies.append(pltpu.async_copy(
                    pred_hbm.at[pl.ds(
                        (_TCB * _NC + c) * _GPIX + pixbase, _RND)],
                    cls_v.at[pl.ds(c * _RND, _RND)], sem))
            for c in range(_NC):
                copies[c].wait()

            @plsc.parallel_loop(0, _RND, step=_LANES, unroll=2)
            def _amax(i):
                best = cls_v[pl.ds(i, _LANES)]
                bidx = zeros
                for c in range(1, _NC):
                    xc = cls_v[pl.ds(c * _RND + i, _LANES)]
                    take = xc > best
                    best = jnp.where(take, xc, best)
                    bidx = jnp.where(take, jnp.full((_LANES,), c, jnp.int32),
                                     bidx)
                comb = lab_v[pl.ds(rnd * _RND + i, _LANES)] * _NC + bidx
                plsc.addupdate_scatter(h16_v, [lane_base + comb], ones)

        _hist_zero_and_reduce_defs(h16_v, hsum_v, zeros)
        pltpu.async_copy(hsum_v, out_hbm.at[wid], sem).wait()

    return amh_kernel(predf, labelf)


def _sc_hist(flat):

    @functools.partial(
        pl.kernel,
        compiler_params=_sc_compiler_params(),
        out_type=jax.ShapeDtypeStruct((_NW, _HPAD), jnp.int32),
        mesh=_MESH,
        scratch_types=[
            pltpu.VMEM((_CHUNK_B,), jnp.int32),
            pltpu.VMEM((_LANES * _HPAD,), jnp.int32),
            pltpu.VMEM((_HPAD,), jnp.int32),
            pltpu.SemaphoreType.DMA,
        ],
    )
    def hist_kernel(flat_hbm, out_hbm, idx_v, h16_v, hsum_v, sem):
        wid = lax.axis_index("s") * 2 + lax.axis_index("c")
        base = wid * _CHUNK_B
        cpy = pltpu.async_copy(flat_hbm.at[pl.ds(base, _CHUNK_B)], idx_v, sem)

        lane = lax.iota(jnp.int32, _LANES)
        lane_base = lane * _HPAD
        ones = jnp.ones((_LANES,), jnp.int32)
        zeros = jnp.zeros((_LANES,), jnp.int32)

        @plsc.parallel_loop(0, _LANES * _HPAD, step=_LANES, unroll=8)
        def _zero(i):
            h16_v[pl.ds(i, _LANES)] = zeros

        cpy.wait()
        _hist_accumulate(h16_v, lane_base, ones, idx_v, _CHUNK_B)
        _hist_zero_and_reduce_defs(h16_v, hsum_v, zeros)
        pltpu.async_copy(hsum_v, out_hbm.at[wid], sem).wait()

    return hist_kernel(flat)


def _finalize_body(part_ref, out_ref):
    h = jnp.sum(part_ref[...].astype(jnp.float32), axis=0)
    ri = lax.broadcasted_iota(jnp.int32, (_NC, _NC), 0)
    ci = lax.broadcasted_iota(jnp.int32, (_NC, _NC), 1)
    eye = ri == ci
    d = jnp.sum(jnp.where(eye, h, 0.0), axis=1)
    row = jnp.sum(h, axis=1)
    col = jnp.sum(h, axis=0)
    denom = row + col - d
    idx = lax.iota(jnp.int32, _NC)
    valid = (denom > 0.0) & (idx >= 1)
    iou = jnp.where(valid, d / jnp.where(denom > 0.0, denom, 1.0), 0.0)
    cnt = jnp.sum(valid.astype(jnp.float32))
    out_ref[...] = (1.0 - jnp.sum(iou) / cnt).reshape(1, 1)


def _finalize(partials):
    return pl.pallas_call(
        _finalize_body,
        out_shape=jax.ShapeDtypeStruct((1, 1), jnp.float32),
    )(partials)


@jax.jit
def kernel(pred, label):
    label = label.astype(jnp.int32)
    parts_a = _sc_argmax_hist(pred.reshape(_B * _NC * _GPIX),
                              label.reshape(_B * _GPIX))
    comb = _combined_index(pred, label)
    parts_b = _sc_hist(comb.reshape(_TC_PIX))
    partials = jnp.concatenate([parts_a, parts_b], axis=0)
    out = _finalize(partials[:, : _NC * _NC].reshape(2 * _NW, _NC, _NC))
    return out[0, 0]


# speedup vs baseline: 2.1670x; 2.1670x over previous
import dataclasses
import functools

import jax
import jax.numpy as jnp
from jax import lax
from jax.experimental import pallas as pl
from jax.experimental.pallas import tpu as pltpu
from jax.experimental.pallas import tpu_sc as plsc

_NC = 19
_R = 128
_H = 512
_W = 512
_B = 4

_NPIX = _B * _H * _W
_NW = 32
_CHUNK = _NPIX // _NW
_HPAD = 368
_LANES = 16


def _argmax_body(pred_ref, label_ref, comb_ref):
    x = pred_ref[0]
    best = x[0]
    bidx = jnp.zeros((_R, _W), jnp.int32)
    for c in range(1, _NC):
        xc = x[c]
        take = xc > best
        best = jnp.where(take, xc, best)
        bidx = jnp.where(take, c, bidx)
    comb_ref[0] = label_ref[0] * _NC + bidx


def _combined_index(pred, label):
    return pl.pallas_call(
        _argmax_body,
        grid=(_B, _H // _R),
        in_specs=[
            pl.BlockSpec((1, _NC, _R, _W), lambda b, r: (b, 0, r, 0)),
            pl.BlockSpec((1, _R, _W), lambda b, r: (b, r, 0)),
        ],
        out_specs=pl.BlockSpec((1, _R, _W), lambda b, r: (b, r, 0)),
        out_shape=jax.ShapeDtypeStruct((_B, _H, _W), jnp.int32),
    )(pred, label)


def _sc_hist(flat):
    mesh = plsc.VectorSubcoreMesh(core_axis_name="c", subcore_axis_name="s")
    cp = pltpu.CompilerParams()
    if "needs_layout_passes" in pltpu.CompilerParams.__dataclass_fields__:
        cp = dataclasses.replace(cp, needs_layout_passes=False)

    @functools.partial(
        pl.kernel,
        compiler_params=cp,
        out_type=jax.ShapeDtypeStruct((_NW, _HPAD), jnp.int32),
        mesh=mesh,
        scratch_types=[
            pltpu.VMEM((_CHUNK,), jnp.int32),
            pltpu.VMEM((_LANES * _HPAD,), jnp.int32),
            pltpu.VMEM((_HPAD,), jnp.int32),
            pltpu.SemaphoreType.DMA,
        ],
    )
    def hist_kernel(flat_hbm, out_hbm, idx_v, h16_v, hsum_v, sem):
        wid = lax.axis_index("s") * 2 + lax.axis_index("c")
        base = wid * _CHUNK
        cp = pltpu.async_copy(flat_hbm.at[pl.ds(base, _CHUNK)], idx_v, sem)

        lane = lax.iota(jnp.int32, _LANES)
        lane_base = lane * _HPAD
        ones = jnp.ones((_LANES,), jnp.int32)
        zeros = jnp.zeros((_LANES,), jnp.int32)

        @plsc.parallel_loop(0, _LANES * _HPAD, step=_LANES, unroll=8)
        def _zero(i):
            h16_v[pl.ds(i, _LANES)] = zeros

        cp.wait()

        @plsc.parallel_loop(0, _CHUNK, step=_LANES, unroll=8)
        def _accum(i):
            v = idx_v[pl.ds(i, _LANES)]
            plsc.addupdate_scatter(h16_v, [lane_base + v], ones)

        @plsc.parallel_loop(0, _HPAD, step=_LANES, unroll=2)
        def _reduce(c):
            acc = zeros
            for l in range(_LANES):
                acc = acc + h16_v[pl.ds(l * _HPAD + c, _LANES)]
            hsum_v[pl.ds(c, _LANES)] = acc

        pltpu.async_copy(hsum_v, out_hbm.at[wid], sem).wait()

    return hist_kernel(flat)


def _finalize_body(part_ref, out_ref):
    h = jnp.sum(part_ref[...].astype(jnp.float32), axis=0)
    ri = lax.broadcasted_iota(jnp.int32, (_NC, _NC), 0)
    ci = lax.broadcasted_iota(jnp.int32, (_NC, _NC), 1)
    eye = ri == ci
    d = jnp.sum(jnp.where(eye, h, 0.0), axis=1)
    row = jnp.sum(h, axis=1)
    col = jnp.sum(h, axis=0)
    denom = row + col - d
    idx = lax.iota(jnp.int32, _NC)
    valid = (denom > 0.0) & (idx >= 1)
    iou = jnp.where(valid, d / jnp.where(denom > 0.0, denom, 1.0), 0.0)
    cnt = jnp.sum(valid.astype(jnp.float32))
    out_ref[...] = (1.0 - jnp.sum(iou) / cnt).reshape(1, 1)


def _finalize(partials):
    return pl.pallas_call(
        _finalize_body,
        out_shape=jax.ShapeDtypeStruct((1, 1), jnp.float32),
    )(partials)


@jax.jit
def kernel(pred, label):
    label = label.astype(jnp.int32)
    comb = _combined_index(pred, label)
    partials = _sc_hist(comb.reshape(_NPIX))
    out = _finalize(partials[:, : _NC * _NC].reshape(_NW, _NC, _NC))
    return out[0, 0]


# speedup vs baseline: 2.1711x; 1.0019x over previous
import dataclasses
import functools

import jax
import jax.numpy as jnp
from jax import lax
from jax.experimental import pallas as pl
from jax.experimental.pallas import tpu as pltpu
from jax.experimental.pallas import tpu_sc as plsc

_NC = 19
_R = 128
_H = 512
_W = 512
_B = 4

_NPIX = _B * _H * _W
_NW = 32
_CHUNK = _NPIX // _NW
_HPAD = 368
_LANES = 16


def _argmax_body(pred_ref, label_ref, comb_ref):
    x = pred_ref[0]
    best = x[0]
    bidx = jnp.zeros((_R, _W), jnp.int32)
    for c in range(1, _NC):
        xc = x[c]
        take = xc > best
        best = jnp.where(take, xc, best)
        bidx = jnp.where(take, c, bidx)
    comb_ref[0] = label_ref[0] * _NC + bidx


def _combined_index(pred, label):
    return pl.pallas_call(
        _argmax_body,
        grid=(_B, _H // _R),
        in_specs=[
            pl.BlockSpec((1, _NC, _R, _W), lambda b, r: (b, 0, r, 0)),
            pl.BlockSpec((1, _R, _W), lambda b, r: (b, r, 0)),
        ],
        out_specs=pl.BlockSpec((1, _R, _W), lambda b, r: (b, r, 0)),
        out_shape=jax.ShapeDtypeStruct((_B, _H, _W), jnp.int32),
    )(pred, label)


def _sc_hist(flat):
    mesh = plsc.VectorSubcoreMesh(core_axis_name="c", subcore_axis_name="s")
    cp = pltpu.CompilerParams()
    if "needs_layout_passes" in pltpu.CompilerParams.__dataclass_fields__:
        cp = dataclasses.replace(cp, needs_layout_passes=False)

    @functools.partial(
        pl.kernel,
        compiler_params=cp,
        out_type=jax.ShapeDtypeStruct((_NW, _HPAD), jnp.int32),
        mesh=mesh,
        scratch_types=[
            pltpu.VMEM((_CHUNK,), jnp.int32),
            pltpu.VMEM((_LANES * _HPAD,), jnp.int32),
            pltpu.VMEM((_HPAD,), jnp.int32),
            pltpu.SemaphoreType.DMA,
            pltpu.SemaphoreType.DMA,
        ],
    )
    def hist_kernel(flat_hbm, out_hbm, idx_v, h16_v, hsum_v, sem, sem2):
        wid = lax.axis_index("s") * 2 + lax.axis_index("c")
        base = wid * _CHUNK
        half = _CHUNK // 2
        cp1 = pltpu.async_copy(
            flat_hbm.at[pl.ds(base, half)], idx_v.at[pl.ds(0, half)], sem)
        cp2 = pltpu.async_copy(
            flat_hbm.at[pl.ds(base + half, half)],
            idx_v.at[pl.ds(half, half)], sem2)

        lane = lax.iota(jnp.int32, _LANES)
        lane_base = lane * _HPAD
        ones = jnp.ones((_LANES,), jnp.int32)
        zeros = jnp.zeros((_LANES,), jnp.int32)

        @plsc.parallel_loop(0, _LANES * _HPAD, step=_LANES, unroll=8)
        def _zero(i):
            h16_v[pl.ds(i, _LANES)] = zeros

        cp1.wait()

        @plsc.parallel_loop(0, half, step=_LANES, unroll=8)
        def _accum1(i):
            v = idx_v[pl.ds(i, _LANES)]
            plsc.addupdate_scatter(h16_v, [lane_base + v], ones)

        cp2.wait()

        @plsc.parallel_loop(half, _CHUNK, step=_LANES, unroll=8)
        def _accum2(i):
            v = idx_v[pl.ds(i, _LANES)]
            plsc.addupdate_scatter(h16_v, [lane_base + v], ones)

        @plsc.parallel_loop(0, _HPAD, step=_LANES, unroll=2)
        def _reduce(c):
            acc = zeros
            for l in range(_LANES):
                acc = acc + h16_v[pl.ds(l * _HPAD + c, _LANES)]
            hsum_v[pl.ds(c, _LANES)] = acc

        pltpu.async_copy(hsum_v, out_hbm.at[wid], sem).wait()

    return hist_kernel(flat)


def _finalize_body(part_ref, out_ref):
    h = jnp.sum(part_ref[...].astype(jnp.float32), axis=0)
    ri = lax.broadcasted_iota(jnp.int32, (_NC, _NC), 0)
    ci = lax.broadcasted_iota(jnp.int32, (_NC, _NC), 1)
    eye = ri == ci
    d = jnp.sum(jnp.where(eye, h, 0.0), axis=1)
    row = jnp.sum(h, axis=1)
    col = jnp.sum(h, axis=0)
    denom = row + col - d
    idx = lax.iota(jnp.int32, _NC)
    valid = (denom > 0.0) & (idx >= 1)
    iou = jnp.where(valid, d / jnp.where(denom > 0.0, denom, 1.0), 0.0)
    cnt = jnp.sum(valid.astype(jnp.float32))
    out_ref[...] = (1.0 - jnp.sum(iou) / cnt).reshape(1, 1)


def _finalize(partials):
    return pl.pallas_call(
        _finalize_body,
        out_shape=jax.ShapeDtypeStruct((1, 1), jnp.float32),
    )(partials)


@jax.jit
def kernel(pred, label):
    label = label.astype(jnp.int32)
    comb = _combined_index(pred, label)
    partials = _sc_hist(comb.reshape(_NPIX))
    out = _finalize(partials[:, : _NC * _NC].reshape(_NW, _NC, _NC))
    return out[0, 0]
